# baseline (speedup 1.0000x reference)
import jax
import jax.numpy as jnp
from jax.experimental import pallas as pl
from jax.experimental.pallas import tpu as pltpu

_R = 512
_C = 4096
_S = 512


def _body(x_ref, tri_ref, o_ref, carry_ref):
    j = pl.program_id(1)

    @pl.when(j == 0)
    def _():
        carry_ref[...] = jnp.zeros_like(carry_ref)

    carry = carry_ref[...]
    tri = tri_ref[...]
    for s in range(_C // _S):
        y = jax.lax.dot(x_ref[:, s * _S:(s + 1) * _S], tri,
                        preferred_element_type=jnp.float32)
        y = y + carry
        o_ref[:, s * _S:(s + 1) * _S] = y
        carry = y[:, _S - 1:_S]
    carry_ref[...] = carry


def kernel(x):
    M, N = x.shape
    tri = jnp.triu(jnp.ones((_S, _S), jnp.float32))
    return pl.pallas_call(
        _body,
        grid=(M // _R, N // _C),
        in_specs=[
            pl.BlockSpec((_R, _C), lambda i, j: (i, j)),
            pl.BlockSpec((_S, _S), lambda i, j: (0, 0)),
        ],
        out_specs=pl.BlockSpec((_R, _C), lambda i, j: (i, j)),
        out_shape=jax.ShapeDtypeStruct((M, N), x.dtype),
        scratch_shapes=[pltpu.VMEM((_R, 1), jnp.float32)],
        compiler_params=pltpu.CompilerParams(
            dimension_semantics=("parallel", "arbitrary"),
        ),
    )(x, tri)


# speedup vs baseline: 1.0056x; 1.0056x over previous
import jax
import jax.numpy as jnp
from jax.experimental import pallas as pl
from jax.experimental.pallas import tpu as pltpu

_R = 256
_C = 8192
_S = 256


def _body(x_ref, tri_ref, o_ref, carry_ref):
    j = pl.program_id(1)

    @pl.when(j == 0)
    def _():
        carry_ref[...] = jnp.zeros_like(carry_ref)

    carry = carry_ref[...]
    tri = tri_ref[...]
    for s in range(_C // _S):
        y = jax.lax.dot(x_ref[:, s * _S:(s + 1) * _S], tri,
                        preferred_element_type=jnp.float32)
        y = y + carry
        o_ref[:, s * _S:(s + 1) * _S] = y
        carry = y[:, _S - 1:_S]
    carry_ref[...] = carry


def kernel(x):
    M, N = x.shape
    tri = jnp.triu(jnp.ones((_S, _S), jnp.float32))
    return pl.pallas_call(
        _body,
        grid=(M // _R, N // _C),
        in_specs=[
            pl.BlockSpec((_R, _C), lambda i, j: (i, j)),
            pl.BlockSpec((_S, _S), lambda i, j: (0, 0)),
        ],
        out_specs=pl.BlockSpec((_R, _C), lambda i, j: (i, j)),
        out_shape=jax.ShapeDtypeStruct((M, N), x.dtype),
        scratch_shapes=[pltpu.VMEM((_R, 1), jnp.float32)],
        compiler_params=pltpu.CompilerParams(
            dimension_semantics=("parallel", "arbitrary"),
        ),
    )(x, tri)


# speedup vs baseline: 1.0314x; 1.0257x over previous
import jax
import jax.numpy as jnp
from jax.experimental import pallas as pl
from jax.experimental.pallas import tpu as pltpu

_R = 512
_C = 4096
_S = 256


def _body(x_ref, tri_ref, o_ref, carry_ref):
    j = pl.program_id(1)

    @pl.when(j == 0)
    def _():
        carry_ref[...] = jnp.zeros_like(carry_ref)

    carry = carry_ref[...]
    tri = tri_ref[...]
    for s in range(_C // _S):
        y = jax.lax.dot(x_ref[:, s * _S:(s + 1) * _S], tri,
                        preferred_element_type=jnp.float32)
        y = y + carry
        o_ref[:, s * _S:(s + 1) * _S] = y
        carry = y[:, _S - 1:_S]
    carry_ref[...] = carry


def kernel(x):
    M, N = x.shape
    tri = jnp.triu(jnp.ones((_S, _S), jnp.float32))
    return pl.pallas_call(
        _body,
        grid=(M // _R, N // _C),
        in_specs=[
            pl.BlockSpec((_R, _C), lambda i, j: (i, j)),
            pl.BlockSpec((_S, _S), lambda i, j: (0, 0)),
        ],
        out_specs=pl.BlockSpec((_R, _C), lambda i, j: (i, j)),
        out_shape=jax.ShapeDtypeStruct((M, N), x.dtype),
        scratch_shapes=[pltpu.VMEM((_R, 1), jnp.float32)],
        compiler_params=pltpu.CompilerParams(
            dimension_semantics=("parallel", "arbitrary"),
        ),
    )(x, tri)
